# Initial kernel scaffold; baseline (speedup 1.0000x reference)
import functools

import jax
import jax.numpy as jnp
from jax import lax
from jax.experimental import pallas as pl
from jax.experimental.pallas import tpu as pltpu
from jax.experimental.pallas import tpu_sc as plsc

N = 10000
D = 128
E = 320000
NC = 2
NS = 16
NW = NC * NS
K = 80
NB = E // (NW * K)
RPT = N // NS
ZR = 125

_MESH = plsc.VectorSubcoreMesh(core_axis_name="c", subcore_axis_name="s")


@functools.partial(
    pl.kernel,
    mesh=_MESH,
    out_type=jax.ShapeDtypeStruct((NC, N, 16), jnp.float32),
    scratch_types=[
        pltpu.VMEM((NB, K), jnp.int32),
        pltpu.VMEM((K, 16), jnp.float32),
        pltpu.VMEM((RPT, 16), jnp.float32),
        pltpu.VMEM_SHARED((N, 16), jnp.float32),
    ],
)
def _sc_degree(dst_hbm, out_hbm, dstv, onesv, zbuf, acc):
    c = lax.axis_index("c")
    s = lax.axis_index("s")
    wid = c * NS + s

    def _fill(i, _):
        zbuf[i, :] = jnp.zeros((16,), jnp.float32)
        return 0

    lax.fori_loop(0, RPT, _fill, 0)

    def _fill1(i, _):
        onesv[i, :] = jnp.ones((16,), jnp.float32)
        return 0

    lax.fori_loop(0, K, _fill1, 0)

    pltpu.sync_copy(zbuf, acc.at[pl.ds(s * RPT, RPT)])
    pltpu.sync_copy(dst_hbm.at[wid], dstv)
    plsc.subcore_barrier()

    def _body(j, _):
        pltpu.sync_copy(onesv, acc.at[dstv.at[j]], add=True)
        return 0

    lax.fori_loop(0, NB, _body, 0)

    plsc.subcore_barrier()
    pltpu.sync_copy(acc.at[pl.ds(s * RPT, RPT)],
                    out_hbm.at[c, pl.ds(s * RPT, RPT)])


@functools.partial(
    pl.kernel,
    mesh=_MESH,
    out_type=jax.ShapeDtypeStruct((NC, N, D), jnp.float32),
    scratch_types=[
        pltpu.VMEM((NB, K), jnp.int32),
        pltpu.VMEM((NB, K), jnp.int32),
        pltpu.VMEM((K, D), jnp.float32),
        pltpu.VMEM((ZR, D), jnp.float32),
        pltpu.VMEM_SHARED((N, D), jnp.float32),
        pltpu.SemaphoreType.DMA,
    ],
)
def _sc_scatter(g_hbm, src_hbm, dst_hbm, out_hbm, srcv, dstv, rows, zbuf, acc,
                sem):
    c = lax.axis_index("c")
    s = lax.axis_index("s")
    wid = c * NS + s

    def _fill(i, _):
        for t in range(D // 16):
            zbuf[i, pl.ds(t * 16, 16)] = jnp.zeros((16,), jnp.float32)
        return 0

    lax.fori_loop(0, ZR, _fill, 0)
    for t in range(RPT // ZR):
        pltpu.sync_copy(zbuf, acc.at[pl.ds(s * RPT + t * ZR, ZR)])
    pltpu.sync_copy(src_hbm.at[wid], srcv)
    pltpu.sync_copy(dst_hbm.at[wid], dstv)
    plsc.subcore_barrier()

    def _body(j, _):
        pltpu.async_copy(g_hbm.at[srcv.at[j]], rows, sem).wait()
        pltpu.sync_copy(rows, acc.at[dstv.at[j]], add=True)
        return 0

    lax.fori_loop(0, NB, _body, 0)

    plsc.subcore_barrier()
    pltpu.sync_copy(acc.at[pl.ds(s * RPT, RPT)],
                    out_hbm.at[c, pl.ds(s * RPT, RPT)])


BLK = 1000
_GRID = N // BLK


def _tc1_body(cnt_ref, x_ref, w_ref, g_ref, dinv_ref):
    cnt = cnt_ref[0, :, 0:1] + cnt_ref[1, :, 0:1] + 1.0
    dinv = lax.rsqrt(cnt)
    g_ref[...] = dinv * jnp.dot(x_ref[...], w_ref[...],
                                preferred_element_type=jnp.float32)
    dinv_ref[...] = dinv


_tc1 = pl.pallas_call(
    _tc1_body,
    grid=(_GRID,),
    in_specs=[
        pl.BlockSpec((NC, BLK, 16), lambda i: (0, i, 0)),
        pl.BlockSpec((BLK, D), lambda i: (i, 0)),
        pl.BlockSpec((D, D), lambda i: (0, 0)),
    ],
    out_specs=[
        pl.BlockSpec((BLK, D), lambda i: (i, 0)),
        pl.BlockSpec((BLK, 1), lambda i: (i, 0)),
    ],
    out_shape=[
        jax.ShapeDtypeStruct((N, D), jnp.float32),
        jax.ShapeDtypeStruct((N, 1), jnp.float32),
    ],
)


def _tc2_body(s_ref, g1_ref, dinv_ref, b_ref, w_ref, g2_ref):
    dinv = dinv_ref[...]
    h = dinv * (s_ref[0] + s_ref[1] + g1_ref[...]) + b_ref[...]
    g2_ref[...] = dinv * jnp.dot(h, w_ref[...],
                                 preferred_element_type=jnp.float32)


_tc2 = pl.pallas_call(
    _tc2_body,
    grid=(_GRID,),
    in_specs=[
        pl.BlockSpec((NC, BLK, D), lambda i: (0, i, 0)),
        pl.BlockSpec((BLK, D), lambda i: (i, 0)),
        pl.BlockSpec((BLK, 1), lambda i: (i, 0)),
        pl.BlockSpec((1, D), lambda i: (0, 0)),
        pl.BlockSpec((D, D), lambda i: (0, 0)),
    ],
    out_specs=pl.BlockSpec((BLK, D), lambda i: (i, 0)),
    out_shape=jax.ShapeDtypeStruct((N, D), jnp.float32),
)


def _tc3_body(s_ref, g2_ref, dinv_ref, b_ref, o_ref):
    o_ref[...] = (dinv_ref[...] * (s_ref[0] + s_ref[1] + g2_ref[...])
                  + b_ref[...])


_tc3 = pl.pallas_call(
    _tc3_body,
    grid=(_GRID,),
    in_specs=[
        pl.BlockSpec((NC, BLK, D), lambda i: (0, i, 0)),
        pl.BlockSpec((BLK, D), lambda i: (i, 0)),
        pl.BlockSpec((BLK, 1), lambda i: (i, 0)),
        pl.BlockSpec((1, D), lambda i: (0, 0)),
    ],
    out_specs=pl.BlockSpec((BLK, D), lambda i: (i, 0)),
    out_shape=jax.ShapeDtypeStruct((N, D), jnp.float32),
)


@jax.jit
def kernel(x, edge_index, W1, b1, W2, b2):
    src = edge_index[0].reshape(NW, NB, K)
    dst = edge_index[1].reshape(NW, NB, K)
    counts = _sc_degree(dst)
    g1, dinv = _tc1(counts, x, W1)
    s1 = _sc_scatter(g1, src, dst)
    g2 = _tc2(s1, g1, dinv, b1.reshape(1, D), W2)
    s2 = _sc_scatter(g2, src, dst)
    return _tc3(s2, g2, dinv, b2.reshape(1, D))


# speedup vs baseline: 14.1672x; 14.1672x over previous
import functools

import jax
import jax.numpy as jnp
from jax import lax
from jax.experimental import pallas as pl
from jax.experimental.pallas import tpu as pltpu
from jax.experimental.pallas import tpu_sc as plsc

N = 10000
D = 128
F = 64
E = 320000
NC = 2
NS = 16
NW = NC * NS
K = 80
NB = E // (NW * K)
EPT = E // NW
NPAD = 10240
RPT = N // NS
ZR = 125

_MESH = plsc.VectorSubcoreMesh(core_axis_name="c", subcore_axis_name="s")


@functools.partial(
    pl.kernel,
    mesh=_MESH,
    out_type=jax.ShapeDtypeStruct((NW, NPAD), jnp.float32),
    scratch_types=[
        pltpu.VMEM((EPT,), jnp.int32),
        pltpu.VMEM((NPAD,), jnp.float32),
    ],
    compiler_params=pltpu.CompilerParams(
        use_tc_tiling_on_sc=False, needs_layout_passes=False
    ),
)
def _sc_degree(dst_hbm, out_hbm, dstv, hist):
    c = lax.axis_index("c")
    s = lax.axis_index("s")
    wid = c * NS + s

    def _zero(i, _):
        hist[pl.ds(i * 16, 16)] = jnp.zeros((16,), jnp.float32)
        return 0

    lax.fori_loop(0, NPAD // 16, _zero, 0)
    pltpu.sync_copy(dst_hbm.at[wid], dstv)

    ones = jnp.ones((16,), jnp.float32)

    def _body(i, _):
        idx = dstv[pl.ds(i * 16, 16)]
        plsc.addupdate_scatter(hist, [idx], ones)
        return 0

    lax.fori_loop(0, EPT // 16, _body, 0)
    pltpu.sync_copy(hist, out_hbm.at[wid])


@functools.partial(
    pl.kernel,
    mesh=_MESH,
    out_type=[
        jax.ShapeDtypeStruct((NC, N, F), jnp.float32),
        jax.ShapeDtypeStruct((NC, N, F), jnp.float32),
    ],
    scratch_types=[
        pltpu.VMEM((NB, K), jnp.int32),
        pltpu.VMEM((NB, K), jnp.int32),
        pltpu.VMEM((K, F), jnp.float32),
        pltpu.VMEM((ZR, F), jnp.float32),
        pltpu.VMEM_SHARED((N, F), jnp.float32),
        pltpu.SemaphoreType.DMA,
    ],
    compiler_params=pltpu.CompilerParams(use_tc_tiling_on_sc=False),
)
def _sc_scatter(ga_hbm, gb_hbm, src_hbm, dst_hbm, outa_hbm, outb_hbm,
                srcv, dstv, rows, zbuf, acc, sem):
    c = lax.axis_index("c")
    s = lax.axis_index("s")
    wid = c * NS + s
    row0 = s * RPT

    def _fill(i, _):
        for t in range(F // 16):
            zbuf[i, pl.ds(t * 16, 16)] = jnp.zeros((16,), jnp.float32)
        return 0

    lax.fori_loop(0, ZR, _fill, 0)

    def _zero_acc():
        for t in range(RPT // ZR):
            pltpu.sync_copy(zbuf, acc.at[pl.ds(row0 + t * ZR, ZR)])

    def _copy_out(dst_ref):
        for t in range(RPT // ZR):
            pltpu.sync_copy(acc.at[pl.ds(row0 + t * ZR, ZR)],
                            dst_ref.at[c, pl.ds(row0 + t * ZR, ZR)])

    def _accumulate(g_ref):
        def _body(j, _):
            pltpu.async_copy(g_ref.at[srcv.at[j]], rows, sem).wait()
            pltpu.sync_copy(rows, acc.at[dstv.at[j]], add=True)
            return 0

        lax.fori_loop(0, NB, _body, 0)

    _zero_acc()
    pltpu.sync_copy(src_hbm.at[wid], srcv)
    pltpu.sync_copy(dst_hbm.at[wid], dstv)
    plsc.subcore_barrier()

    _accumulate(ga_hbm)
    plsc.subcore_barrier()
    _copy_out(outa_hbm)
    _zero_acc()
    plsc.subcore_barrier()

    _accumulate(gb_hbm)
    plsc.subcore_barrier()
    _copy_out(outb_hbm)


def _tc0_body(cnt_ref, dinv_ref):
    deg = jnp.sum(cnt_ref[...], axis=0) + 1.0
    dinv_ref[...] = lax.rsqrt(deg)


_tc0 = pl.pallas_call(
    _tc0_body,
    out_shape=jax.ShapeDtypeStruct((NPAD,), jnp.float32),
)


BLK = 1000
_GRID = N // BLK


def _tc1_body(dinv_ref, x_ref, w_ref, ga_ref, gb_ref):
    g = dinv_ref[...] * jnp.dot(x_ref[...], w_ref[...],
                                preferred_element_type=jnp.float32)
    ga_ref[...] = g[:, :F]
    gb_ref[...] = g[:, F:]


_tc1 = pl.pallas_call(
    _tc1_body,
    grid=(_GRID,),
    in_specs=[
        pl.BlockSpec((BLK, 1), lambda i: (i, 0)),
        pl.BlockSpec((BLK, D), lambda i: (i, 0)),
        pl.BlockSpec((D, D), lambda i: (0, 0)),
    ],
    out_specs=[
        pl.BlockSpec((BLK, F), lambda i: (i, 0)),
        pl.BlockSpec((BLK, F), lambda i: (i, 0)),
    ],
    out_shape=[
        jax.ShapeDtypeStruct((N, F), jnp.float32),
        jax.ShapeDtypeStruct((N, F), jnp.float32),
    ],
)


def _tc2_body(sa_ref, sb_ref, ga_ref, gb_ref, dinv_ref, b_ref, w_ref,
              g2a_ref, g2b_ref):
    dinv = dinv_ref[...]
    ha = sa_ref[0] + sa_ref[1] + ga_ref[...]
    hb = sb_ref[0] + sb_ref[1] + gb_ref[...]
    h = dinv * jnp.concatenate([ha, hb], axis=1) + b_ref[...]
    g2 = dinv * jnp.dot(h, w_ref[...], preferred_element_type=jnp.float32)
    g2a_ref[...] = g2[:, :F]
    g2b_ref[...] = g2[:, F:]


_tc2 = pl.pallas_call(
    _tc2_body,
    grid=(_GRID,),
    in_specs=[
        pl.BlockSpec((NC, BLK, F), lambda i: (0, i, 0)),
        pl.BlockSpec((NC, BLK, F), lambda i: (0, i, 0)),
        pl.BlockSpec((BLK, F), lambda i: (i, 0)),
        pl.BlockSpec((BLK, F), lambda i: (i, 0)),
        pl.BlockSpec((BLK, 1), lambda i: (i, 0)),
        pl.BlockSpec((1, D), lambda i: (0, 0)),
        pl.BlockSpec((D, D), lambda i: (0, 0)),
    ],
    out_specs=[
        pl.BlockSpec((BLK, F), lambda i: (i, 0)),
        pl.BlockSpec((BLK, F), lambda i: (i, 0)),
    ],
    out_shape=[
        jax.ShapeDtypeStruct((N, F), jnp.float32),
        jax.ShapeDtypeStruct((N, F), jnp.float32),
    ],
)


def _tc3_body(sa_ref, sb_ref, g2a_ref, g2b_ref, dinv_ref, b_ref, o_ref):
    ha = sa_ref[0] + sa_ref[1] + g2a_ref[...]
    hb = sb_ref[0] + sb_ref[1] + g2b_ref[...]
    o_ref[...] = (dinv_ref[...] * jnp.concatenate([ha, hb], axis=1)
                  + b_ref[...])


_tc3 = pl.pallas_call(
    _tc3_body,
    grid=(_GRID,),
    in_specs=[
        pl.BlockSpec((NC, BLK, F), lambda i: (0, i, 0)),
        pl.BlockSpec((NC, BLK, F), lambda i: (0, i, 0)),
        pl.BlockSpec((BLK, F), lambda i: (i, 0)),
        pl.BlockSpec((BLK, F), lambda i: (i, 0)),
        pl.BlockSpec((BLK, 1), lambda i: (i, 0)),
        pl.BlockSpec((1, D), lambda i: (0, 0)),
    ],
    out_specs=pl.BlockSpec((BLK, D), lambda i: (i, 0)),
    out_shape=jax.ShapeDtypeStruct((N, D), jnp.float32),
)


@jax.jit
def kernel(x, edge_index, W1, b1, W2, b2):
    src = edge_index[0].reshape(NW, NB, K)
    dst = edge_index[1].reshape(NW, NB, K)
    dst_flat = edge_index[1].reshape(NW, EPT)
    counts = _sc_degree(dst_flat)
    dinv = _tc0(counts)[:N].reshape(N, 1)
    g1a, g1b = _tc1(dinv, x, W1)
    s1a, s1b = _sc_scatter(g1a, g1b, src, dst)
    g2a, g2b = _tc2(s1a, s1b, g1a, g1b, dinv, b1.reshape(1, D), W2)
    s2a, s2b = _sc_scatter(g2a, g2b, src, dst)
    return _tc3(s2a, s2b, g2a, g2b, dinv, b2.reshape(1, D))
